# Initial kernel scaffold; baseline (speedup 1.0000x reference)
import jax
import jax.numpy as jnp
from jax.experimental import pallas as pl


def kernel(x, edge_index, W1, b1, W2, b2, W3, b3, fcW, fcb, fc2W, fc2b, fc3W, fc3b):
    raise NotImplementedError("write your pallas kernel here")


# speedup vs baseline: 28.4215x; 28.4215x over previous
import functools

import jax
import jax.numpy as jnp
from jax import lax
from jax.experimental import pallas as pl
from jax.experimental.pallas import tpu as pltpu
from jax.experimental.pallas import tpu_sc as plsc

NC = 2
NS = 16
NW = NC * NS
WIN = 128


def _mesh():
  return plsc.VectorSubcoreMesh(
      core_axis_name="c", subcore_axis_name="s", num_cores=NC, num_subcores=NS
  )


def _make_edge_pass(n_pad, nwin):
  rpt = n_pad // NS

  @functools.partial(
      pl.kernel,
      out_type=jax.ShapeDtypeStruct((NC, n_pad, 16), jnp.float32),
      mesh=_mesh(),
      scratch_types=[
          pltpu.VMEM((nwin, WIN), jnp.int32),
          pltpu.VMEM((nwin, WIN), jnp.int32),
          pltpu.VMEM((WIN, 16), jnp.float32),
          pltpu.VMEM_SHARED((n_pad, 16), jnp.float32),
          pltpu.SemaphoreType.DMA,
      ],
      compiler_params=pltpu.CompilerParams(use_tc_tiling_on_sc=False),
  )
  def edge_pass(g_hbm, init_hbm, src_hbm, dst_hbm, out_hbm,
                srcv, dstv, rows, acc, sem):
    c = lax.axis_index("c")
    s = lax.axis_index("s")
    wid = s * NC + c
    pltpu.sync_copy(init_hbm.at[pl.ds(s * rpt, rpt)], acc.at[pl.ds(s * rpt, rpt)])
    pltpu.sync_copy(src_hbm.at[wid], srcv)
    pltpu.sync_copy(dst_hbm.at[wid], dstv)
    plsc.subcore_barrier()

    def body(w, carry):
      pltpu.async_copy(g_hbm.at[srcv.at[w]], rows, sem).wait()
      pltpu.sync_copy(rows, acc.at[dstv.at[w]], add=True)
      return carry

    lax.fori_loop(0, nwin, body, 0)
    plsc.subcore_barrier()
    pltpu.sync_copy(acc.at[pl.ds(s * rpt, rpt)],
                    out_hbm.at[c].at[pl.ds(s * rpt, rpt)])

  return edge_pass


def _make_deg_pass(n_pad, nwin):
  rpt = n_pad // NS

  @functools.partial(
      pl.kernel,
      out_type=jax.ShapeDtypeStruct((NC, n_pad, 16), jnp.float32),
      mesh=_mesh(),
      scratch_types=[
          pltpu.VMEM((nwin, WIN), jnp.int32),
          pltpu.VMEM((WIN, 16), jnp.float32),
          pltpu.VMEM_SHARED((n_pad, 16), jnp.float32),
      ],
      compiler_params=pltpu.CompilerParams(use_tc_tiling_on_sc=False),
  )
  def deg_pass(zeros_hbm, ones_hbm, dst_hbm, out_hbm, dstv, ones_v, acc):
    c = lax.axis_index("c")
    s = lax.axis_index("s")
    wid = s * NC + c
    pltpu.sync_copy(zeros_hbm.at[pl.ds(s * rpt, rpt)], acc.at[pl.ds(s * rpt, rpt)])
    pltpu.sync_copy(ones_hbm, ones_v)
    pltpu.sync_copy(dst_hbm.at[wid], dstv)
    plsc.subcore_barrier()

    def body(w, carry):
      pltpu.sync_copy(ones_v, acc.at[dstv.at[w]], add=True)
      return carry

    lax.fori_loop(0, nwin, body, 0)
    plsc.subcore_barrier()
    pltpu.sync_copy(acc.at[pl.ds(s * rpt, rpt)],
                    out_hbm.at[c].at[pl.ds(s * rpt, rpt)])

  return deg_pass


def _tc_first(n, n_pad):
  def body(p_ref, x_ref, w1_ref, dinv_ref, g_ref):
    deg = p_ref[0] + p_ref[1] + 1.0
    dinv = lax.rsqrt(deg)
    dinv_ref[...] = dinv
    h = jnp.dot(x_ref[...], w1_ref[...], preferred_element_type=jnp.float32)
    g_ref[0:n, :] = h * dinv[0:n, :]
    g_ref[n:n_pad, :] = jnp.zeros((n_pad - n, 16), jnp.float32)

  return pl.pallas_call(
      body,
      out_shape=(
          jax.ShapeDtypeStruct((n_pad, 16), jnp.float32),
          jax.ShapeDtypeStruct((n_pad, 16), jnp.float32),
      ),
  )


def _tc_mid(n, n_pad):
  def body(p_ref, g_ref, dinv_ref, b_ref, w_ref, out_ref):
    t = p_ref[0] + p_ref[1] - g_ref[...]
    h = jnp.tanh(dinv_ref[...] * t + b_ref[...])
    g2 = jnp.dot(h, w_ref[...], preferred_element_type=jnp.float32)
    out_ref[0:n, :] = g2[0:n, :] * dinv_ref[0:n, :]
    out_ref[n:n_pad, :] = jnp.zeros((n_pad - n, 16), jnp.float32)

  return pl.pallas_call(
      body, out_shape=jax.ShapeDtypeStruct((n_pad, 16), jnp.float32)
  )


def _tc_final(n, n_pad, d_out):
  def body(p_ref, g_ref, dinv_ref, b3_ref, fcw_ref, fcb_ref, fc2w_ref,
           fc2b_ref, fc3w_ref, fc3b_ref, out_ref):
    t = p_ref[0] + p_ref[1] - g_ref[...]
    h = jnp.tanh(dinv_ref[...] * t + b3_ref[...])[0:n, :]
    c1 = jnp.maximum(
        jnp.dot(h, fcw_ref[...], preferred_element_type=jnp.float32)
        + fcb_ref[...], 0.0)
    c2 = jnp.maximum(
        jnp.dot(c1, fc2w_ref[...], preferred_element_type=jnp.float32)
        + fc2b_ref[...], 0.0)
    out_ref[...] = (
        jnp.dot(c2, fc3w_ref[...], preferred_element_type=jnp.float32)
        + fc3b_ref[...])

  return pl.pallas_call(
      body, out_shape=jax.ShapeDtypeStruct((n, d_out), jnp.float32)
  )


@jax.jit
def kernel(x, edge_index, W1, b1, W2, b2, W3, b3, fcW, fcb, fc2W, fc2b,
           fc3W, fc3b):
  n = x.shape[1]
  d_out = fc3W.shape[1]
  e = edge_index.shape[1]
  n_pad = -(-(n + 16) // 128) * 128
  epw = -(-e // (NW * WIN)) * WIN
  e_pad = NW * epw
  nwin = epw // WIN

  x2 = x[0]
  pad = e_pad - e
  i = jnp.arange(pad, dtype=jnp.int32)
  src_p = jnp.concatenate([edge_index[0], n + (i % 8)])
  dst_p = jnp.concatenate([edge_index[1], n + 8 + (i % 8)])
  src3 = src_p.reshape(NW, nwin, WIN)
  dst3 = dst_p.reshape(NW, nwin, WIN)

  zeros_np = jnp.zeros((n_pad, 16), jnp.float32)
  ones_w = jnp.ones((WIN, 16), jnp.float32)

  deg_p = _make_deg_pass(n_pad, nwin)(zeros_np, ones_w, dst3)
  dinv, g1 = _tc_first(n, n_pad)(deg_p, x2, W1)

  edge_pass = _make_edge_pass(n_pad, nwin)
  p1 = edge_pass(g1, g1, src3, dst3)
  g2 = _tc_mid(n, n_pad)(p1, g1, dinv, b1.reshape(1, -1), W2)
  p2 = edge_pass(g2, g2, src3, dst3)
  g3 = _tc_mid(n, n_pad)(p2, g2, dinv, b2.reshape(1, -1), W3)
  p3 = edge_pass(g3, g3, src3, dst3)
  out = _tc_final(n, n_pad, d_out)(
      p3, g3, dinv, b3.reshape(1, -1), fcW, fcb.reshape(1, -1),
      fc2W, fc2b.reshape(1, -1), fc3W, fc3b.reshape(1, -1))
  return out[None, :, :]


# speedup vs baseline: 33.6567x; 1.1842x over previous
import functools

import jax
import jax.numpy as jnp
from jax import lax
from jax.experimental import pallas as pl
from jax.experimental.pallas import tpu as pltpu
from jax.experimental.pallas import tpu_sc as plsc

NC = 2
NS = 16
NW = NC * NS
WIN = 128


def _mesh():
  return plsc.VectorSubcoreMesh(
      core_axis_name="c", subcore_axis_name="s", num_cores=NC, num_subcores=NS
  )


K = 4


def _make_edge_pass(n_pad, nwin):
  rpt = n_pad // NS
  assert nwin % (2 * K) == 0

  @functools.partial(
      pl.kernel,
      out_type=jax.ShapeDtypeStruct((NC, n_pad, 16), jnp.float32),
      mesh=_mesh(),
      scratch_types=[
          pltpu.VMEM((nwin, WIN), jnp.int32),
          pltpu.VMEM((nwin, WIN), jnp.int32),
          pltpu.VMEM((2 * K, WIN, 16), jnp.float32),
          pltpu.VMEM_SHARED((n_pad, 16), jnp.float32),
          pltpu.SemaphoreType.DMA((2 * K,)),
          pltpu.SemaphoreType.DMA((2 * K,)),
      ],
      compiler_params=pltpu.CompilerParams(use_tc_tiling_on_sc=False),
  )
  def edge_pass(g_hbm, init_hbm, src_hbm, dst_hbm, out_hbm,
                srcv, dstv, rows, acc, gsem, ssem):
    c = lax.axis_index("c")
    s = lax.axis_index("s")
    wid = s * NC + c
    pltpu.sync_copy(init_hbm.at[pl.ds(s * rpt, rpt)], acc.at[pl.ds(s * rpt, rpt)])
    pltpu.sync_copy(src_hbm.at[wid], srcv)
    pltpu.sync_copy(dst_hbm.at[wid], dstv)
    plsc.subcore_barrier()

    def gather(w, b):
      pltpu.async_copy(g_hbm.at[srcv.at[w]], rows.at[b], gsem.at[b])

    def wait_gather(b):
      pltpu.make_async_copy(g_hbm.at[pl.ds(0, WIN)], rows.at[b], gsem.at[b]).wait()

    def scatter(w, b):
      pltpu.async_copy(rows.at[b], acc.at[dstv.at[w]], ssem.at[b], add=True)

    def wait_scatter(b):
      pltpu.make_async_copy(g_hbm.at[pl.ds(0, WIN)], rows.at[b], ssem.at[b]).wait()

    for j in range(2 * K):
      gather(j, j)

    def body(ch2, carry):
      even = ch2 * 2 * K
      odd = even + K
      nxt_even = odd + K
      nxt_odd = nxt_even + K
      for j in range(K):
        wait_gather(j)
        scatter(even + j, j)
      for j in range(K):
        wait_scatter(j)

        @pl.when(nxt_even + j < nwin)
        def _():
          gather(nxt_even + j, j)

      for j in range(K):
        wait_gather(K + j)
        scatter(odd + j, K + j)
      for j in range(K):
        wait_scatter(K + j)

        @pl.when(nxt_odd + j < nwin)
        def _():
          gather(nxt_odd + j, K + j)

      return carry

    lax.fori_loop(0, nwin // (2 * K), body, 0)
    plsc.subcore_barrier()
    pltpu.sync_copy(acc.at[pl.ds(s * rpt, rpt)],
                    out_hbm.at[c].at[pl.ds(s * rpt, rpt)])

  return edge_pass


def _make_deg_pass(n_pad, nwin):
  rpt = n_pad // NS

  @functools.partial(
      pl.kernel,
      out_type=jax.ShapeDtypeStruct((NC, n_pad, 16), jnp.float32),
      mesh=_mesh(),
      scratch_types=[
          pltpu.VMEM((nwin, WIN), jnp.int32),
          pltpu.VMEM((WIN, 16), jnp.float32),
          pltpu.VMEM_SHARED((n_pad, 16), jnp.float32),
          pltpu.SemaphoreType.DMA((2 * K,)),
      ],
      compiler_params=pltpu.CompilerParams(use_tc_tiling_on_sc=False),
  )
  def deg_pass(zeros_hbm, ones_hbm, dst_hbm, out_hbm, dstv, ones_v, acc, ssem):
    c = lax.axis_index("c")
    s = lax.axis_index("s")
    wid = s * NC + c
    pltpu.sync_copy(zeros_hbm.at[pl.ds(s * rpt, rpt)], acc.at[pl.ds(s * rpt, rpt)])
    pltpu.sync_copy(ones_hbm, ones_v)
    pltpu.sync_copy(dst_hbm.at[wid], dstv)
    plsc.subcore_barrier()

    def body(ch, carry):
      base = ch * 2 * K
      for j in range(2 * K):
        pltpu.async_copy(ones_v, acc.at[dstv.at[base + j]], ssem.at[j],
                         add=True)
      for j in range(2 * K):
        pltpu.make_async_copy(ones_hbm, ones_v, ssem.at[j]).wait()
      return carry

    lax.fori_loop(0, nwin // (2 * K), body, 0)
    plsc.subcore_barrier()
    pltpu.sync_copy(acc.at[pl.ds(s * rpt, rpt)],
                    out_hbm.at[c].at[pl.ds(s * rpt, rpt)])

  return deg_pass


def _tc_first(n, n_pad):
  def body(p_ref, x_ref, w1_ref, dinv_ref, g_ref):
    deg = p_ref[0] + p_ref[1] + 1.0
    dinv = lax.rsqrt(deg)
    dinv_ref[...] = dinv
    h = jnp.dot(x_ref[...], w1_ref[...], preferred_element_type=jnp.float32)
    g_ref[0:n, :] = h * dinv[0:n, :]
    g_ref[n:n_pad, :] = jnp.zeros((n_pad - n, 16), jnp.float32)

  return pl.pallas_call(
      body,
      out_shape=(
          jax.ShapeDtypeStruct((n_pad, 16), jnp.float32),
          jax.ShapeDtypeStruct((n_pad, 16), jnp.float32),
      ),
  )


def _tc_mid(n, n_pad):
  def body(p_ref, g_ref, dinv_ref, b_ref, w_ref, out_ref):
    t = p_ref[0] + p_ref[1] - g_ref[...]
    h = jnp.tanh(dinv_ref[...] * t + b_ref[...])
    g2 = jnp.dot(h, w_ref[...], preferred_element_type=jnp.float32)
    out_ref[0:n, :] = g2[0:n, :] * dinv_ref[0:n, :]
    out_ref[n:n_pad, :] = jnp.zeros((n_pad - n, 16), jnp.float32)

  return pl.pallas_call(
      body, out_shape=jax.ShapeDtypeStruct((n_pad, 16), jnp.float32)
  )


def _tc_final(n, n_pad, d_out):
  def body(p_ref, g_ref, dinv_ref, b3_ref, fcw_ref, fcb_ref, fc2w_ref,
           fc2b_ref, fc3w_ref, fc3b_ref, out_ref):
    t = p_ref[0] + p_ref[1] - g_ref[...]
    h = jnp.tanh(dinv_ref[...] * t + b3_ref[...])[0:n, :]
    c1 = jnp.maximum(
        jnp.dot(h, fcw_ref[...], preferred_element_type=jnp.float32)
        + fcb_ref[...], 0.0)
    c2 = jnp.maximum(
        jnp.dot(c1, fc2w_ref[...], preferred_element_type=jnp.float32)
        + fc2b_ref[...], 0.0)
    out_ref[...] = (
        jnp.dot(c2, fc3w_ref[...], preferred_element_type=jnp.float32)
        + fc3b_ref[...])

  return pl.pallas_call(
      body, out_shape=jax.ShapeDtypeStruct((n, d_out), jnp.float32)
  )


@jax.jit
def kernel(x, edge_index, W1, b1, W2, b2, W3, b3, fcW, fcb, fc2W, fc2b,
           fc3W, fc3b):
  n = x.shape[1]
  d_out = fc3W.shape[1]
  e = edge_index.shape[1]
  n_pad = -(-(n + 16) // 128) * 128
  epw = -(-e // (NW * WIN * 2 * K)) * WIN * 2 * K
  e_pad = NW * epw
  nwin = epw // WIN

  x2 = x[0]
  pad = e_pad - e
  i = jnp.arange(pad, dtype=jnp.int32)
  src_p = jnp.concatenate([edge_index[0], n + (i % 8)])
  dst_p = jnp.concatenate([edge_index[1], n + 8 + (i % 8)])
  src3 = src_p.reshape(NW, nwin, WIN)
  dst3 = dst_p.reshape(NW, nwin, WIN)

  zeros_np = jnp.zeros((n_pad, 16), jnp.float32)
  ones_w = jnp.ones((WIN, 16), jnp.float32)

  deg_p = _make_deg_pass(n_pad, nwin)(zeros_np, ones_w, dst3)
  dinv, g1 = _tc_first(n, n_pad)(deg_p, x2, W1)

  edge_pass = _make_edge_pass(n_pad, nwin)
  p1 = edge_pass(g1, g1, src3, dst3)
  g2 = _tc_mid(n, n_pad)(p1, g1, dinv, b1.reshape(1, -1), W2)
  p2 = edge_pass(g2, g2, src3, dst3)
  g3 = _tc_mid(n, n_pad)(p2, g2, dinv, b2.reshape(1, -1), W3)
  p3 = edge_pass(g3, g3, src3, dst3)
  out = _tc_final(n, n_pad, d_out)(
      p3, g3, dinv, b3.reshape(1, -1), fcW, fcb.reshape(1, -1),
      fc2W, fc2b.reshape(1, -1), fc3W, fc3b.reshape(1, -1))
  return out[None, :, :]
